# Initial kernel scaffold; baseline (speedup 1.0000x reference)
import jax
import jax.numpy as jnp
from jax.experimental import pallas as pl


def kernel(prob, label):
    raise NotImplementedError("write your pallas kernel here")


# speedup vs baseline: 17.6220x; 17.6220x over previous
import functools

import jax
import jax.numpy as jnp
from jax.experimental import pallas as pl

_K1 = 6
_BATCH = 16384
_C = 1000
_BLK = 256


def _body(prob_ref, lab_ref, out_ref):
    i = pl.program_id(0)
    x = prob_ref[...]
    lab = lab_ref[0, 0, :]
    neg = jnp.float32(-jnp.inf)

    m = jnp.max(x, axis=1)
    cnt = jnp.sum((x == m[:, None]).astype(jnp.int32), axis=1)
    ans = m
    done = cnt >= _K1
    for _ in range(_K1 - 1):
        nm = jnp.max(jnp.where(x < m[:, None], x, neg), axis=1)
        c2 = cnt + jnp.sum((x == nm[:, None]).astype(jnp.int32), axis=1)
        ans = jnp.where(done, ans, nm)
        m = jnp.where(done, m, nm)
        cnt = jnp.where(done, cnt, c2)
        done = cnt >= _K1

    thresh = ans

    pred_cnt = jnp.sum((x > thresh[:, None]).astype(jnp.int32), axis=1)
    iota = jax.lax.broadcasted_iota(jnp.int32, (_BLK, _C), 1)
    lab_val = jnp.max(jnp.where(iota == lab[:, None], x, neg), axis=1)

    inter = (lab_val > thresh).astype(jnp.int32)
    union = pred_cnt + 1 - inter
    iou = inter.astype(jnp.float32) / union.astype(jnp.float32)
    part = jnp.sum(iou)

    @pl.when(i == 0)
    def _init():
        out_ref[...] = jnp.zeros((1, 1), jnp.float32)

    out_ref[...] = out_ref[...] + part


@jax.jit
def kernel(prob, label):
    nb = _BATCH // _BLK
    lab3 = label.reshape(nb, 1, _BLK)
    out = pl.pallas_call(
        _body,
        grid=(nb,),
        in_specs=[
            pl.BlockSpec((_BLK, _C), lambda i: (i, 0)),
            pl.BlockSpec((1, 1, _BLK), lambda i: (i, 0, 0)),
        ],
        out_specs=pl.BlockSpec((1, 1), lambda i: (0, 0)),
        out_shape=jax.ShapeDtypeStruct((1, 1), jnp.float32),
    )(prob, lab3)
    return out[0, 0] / jnp.float32(_BATCH)
